# baseline (speedup 1.0000x reference)
import functools
import math

import jax
import jax.numpy as jnp
from jax import lax
from jax.experimental import pallas as pl
from jax.experimental.pallas import tpu as pltpu
from jax.experimental.pallas import tpu_sc as plsc

_N_NODES = 10000
_E = 320000
_D = 128
_PROB = _E / (_N_NODES ** 2 - _N_NODES) * 2.0
_EPS = -math.log(1.0 - _PROB)

_NC = 2
_NS = 16
_NW = _NC * _NS
_EPT = _E // _NW
_B = 80
_NB = _EPT // _B
_G = _B // 16
_L = 16


def _edge_partial(srows, trows, row):
    p = [srows[row, pl.ds(j * _L, _L)] * trows[row, pl.ds(j * _L, _L)]
         for j in range(_D // _L)]
    return ((p[0] + p[1]) + (p[2] + p[3])) + ((p[4] + p[5]) + (p[6] + p[7]))


def _sc_body(table_hbm, ps_hbm, pd_hbm, ns_hbm, nd_hbm, pout_hbm, nout_hbm,
             sidx, didx, s0, t0, s1, t1, pb0, pb1, accbuf,
             semA, semB, semP0, semP1):
    wid = lax.axis_index("s") * _NC + lax.axis_index("c")
    base = wid * _EPT

    def issue(b, sbuf, tbuf, sem):
        pltpu.async_copy(table_hbm.at[sidx.at[pl.ds(b * _B, _B)]], sbuf, sem)
        pltpu.async_copy(table_hbm.at[didx.at[pl.ds(b * _B, _B)]], tbuf, sem)

    def wait_pair(b, sbuf, tbuf, sem):
        pltpu.make_async_copy(
            table_hbm.at[sidx.at[pl.ds(b * _B, _B)]], sbuf, sem).wait()
        pltpu.make_async_copy(
            table_hbm.at[didx.at[pl.ds(b * _B, _B)]], tbuf, sem).wait()

    pltpu.sync_copy(ps_hbm.at[pl.ds(base, _EPT)], sidx)
    pltpu.sync_copy(pd_hbm.at[pl.ds(base, _EPT)], didx)
    issue(0, s0, t0, semA)

    def pos_compute(b, sbuf, tbuf, pbuf, psem):
        def group(g, c2):
            for e in range(16):
                pbuf[pl.ds((g * 16 + e) * _L, _L)] = _edge_partial(
                    sbuf, tbuf, g * 16 + e)
            return c2
        lax.fori_loop(0, _G, group, 0)
        pltpu.async_copy(
            pbuf, pout_hbm.at[pl.ds((base + b * _B) * _L, _B * _L)], psem)

    def pos_wait_out(b, pbuf, psem):
        pltpu.make_async_copy(
            pbuf, pout_hbm.at[pl.ds((base + b * _B) * _L, _B * _L)],
            psem).wait()

    def pos_pair(i2, carry):
        b0 = 2 * i2
        issue(b0 + 1, s1, t1, semB)
        wait_pair(b0, s0, t0, semA)

        @pl.when(i2 > 0)
        def _():
            pos_wait_out(b0 - 2, pb0, semP0)
        pos_compute(b0, s0, t0, pb0, semP0)

        issue(b0 + 2, s0, t0, semA)
        wait_pair(b0 + 1, s1, t1, semB)

        @pl.when(i2 > 0)
        def _():
            pos_wait_out(b0 - 1, pb1, semP1)
        pos_compute(b0 + 1, s1, t1, pb1, semP1)
        return carry

    lax.fori_loop(0, (_NB - 1) // 2, pos_pair, 0)
    wait_pair(_NB - 1, s0, t0, semA)
    pos_wait_out(_NB - 3, pb0, semP0)
    pos_compute(_NB - 1, s0, t0, pb0, semP0)
    pos_wait_out(_NB - 2, pb1, semP1)
    pos_wait_out(_NB - 1, pb0, semP0)

    pltpu.sync_copy(ns_hbm.at[pl.ds(base, _EPT)], sidx)
    pltpu.sync_copy(nd_hbm.at[pl.ds(base, _EPT)], didx)
    issue(0, s0, t0, semA)

    def neg_compute(sbuf, tbuf, acc):
        def group(g, bacc):
            for e in range(16):
                bacc = bacc + _edge_partial(sbuf, tbuf, g * 16 + e)
            return bacc
        return acc + lax.fori_loop(0, _G, group, jnp.zeros((_L,), jnp.float32))

    def neg_pair(i2, acc):
        b0 = 2 * i2
        issue(b0 + 1, s1, t1, semB)
        wait_pair(b0, s0, t0, semA)
        acc = neg_compute(s0, t0, acc)
        issue(b0 + 2, s0, t0, semA)
        wait_pair(b0 + 1, s1, t1, semB)
        acc = neg_compute(s1, t1, acc)
        return acc

    acc = lax.fori_loop(0, (_NB - 1) // 2, neg_pair,
                        jnp.zeros((_L,), jnp.float32))
    wait_pair(_NB - 1, s0, t0, semA)
    acc = neg_compute(s0, t0, acc)
    accbuf[...] = acc
    pltpu.sync_copy(accbuf, nout_hbm.at[pl.ds(wid * _L, _L)])


_sc_dots = functools.partial(
    pl.kernel,
    mesh=plsc.VectorSubcoreMesh(core_axis_name="c", subcore_axis_name="s"),
    out_type=(
        jax.ShapeDtypeStruct((_E * _L,), jnp.float32),
        jax.ShapeDtypeStruct((_NW * _L,), jnp.float32),
    ),
    scratch_types=[
        pltpu.VMEM((_EPT,), jnp.int32),
        pltpu.VMEM((_EPT,), jnp.int32),
        pltpu.VMEM((_B, _D), jnp.float32),
        pltpu.VMEM((_B, _D), jnp.float32),
        pltpu.VMEM((_B, _D), jnp.float32),
        pltpu.VMEM((_B, _D), jnp.float32),
        pltpu.VMEM((_B * _L,), jnp.float32),
        pltpu.VMEM((_B * _L,), jnp.float32),
        pltpu.VMEM((_L,), jnp.float32),
        pltpu.SemaphoreType.DMA,
        pltpu.SemaphoreType.DMA,
        pltpu.SemaphoreType.DMA,
        pltpu.SemaphoreType.DMA,
    ],
)(_sc_body)


_ROWS = _E * _L // _D
_BLK = 2000
_NSTEP = _ROWS // _BLK


def _loss_body(pref, nref, oref):
    i = pl.program_id(0)
    part = pref[...]
    k = lax.broadcasted_iota(jnp.int32, (_D, _D), 0)
    j = lax.broadcasted_iota(jnp.int32, (_D, _D), 1)
    seg = jnp.where((j < _D // _L) & (k // _L == j), 1.0, 0.0).astype(jnp.float32)
    dots = jnp.dot(part, seg, preferred_element_type=jnp.float32)
    col = lax.broadcasted_iota(jnp.int32, (_BLK, _D), 1)
    terms = jnp.where(col < _D // _L,
                      jnp.log(1.0 - jnp.exp(-_EPS - dots)), 0.0)
    s = jnp.sum(terms)

    @pl.when(i == 0)
    def _():
        oref[...] = jnp.zeros_like(oref)

    oref[...] = oref[...] + jnp.reshape(s, (1, 1))

    @pl.when(i == _NSTEP - 1)
    def _():
        pos_sum = oref[0, 0]
        neg_sum = jnp.sum(nref[...])
        loss = -pos_sum / _E + neg_sum / _E
        oref[...] = jnp.reshape(loss, (1, 1))


def kernel(block_outputs, pos_edge_index, neg_edge_index):
    pos_i = pos_edge_index.astype(jnp.int32)
    neg_i = neg_edge_index.astype(jnp.int32)
    partials, negacc = _sc_dots(
        block_outputs, pos_i[0], pos_i[1], neg_i[0], neg_i[1])
    loss = pl.pallas_call(
        _loss_body,
        grid=(_NSTEP,),
        in_specs=[
            pl.BlockSpec((_BLK, _D), lambda i: (i, 0)),
            pl.BlockSpec((_NW * _L // _D, _D), lambda i: (0, 0)),
        ],
        out_specs=pl.BlockSpec((1, 1), lambda i: (0, 0)),
        out_shape=jax.ShapeDtypeStruct((1, 1), jnp.float32),
    )(partials.reshape(_ROWS, _D), negacc.reshape(_NW * _L // _D, _D))
    return loss[0, 0]
